# baseline (speedup 1.0000x reference)
import jax
import jax.numpy as jnp
from jax.experimental import pallas as pl
from jax.experimental.pallas import tpu as pltpu

TOP_K = 8
NUM_EXPERTS = 64
D_MODEL = 768
BLOCK_M = 4096
CHUNK = 512


def _gate_kernel(h_ref, w_ref, b_ref, out_w_ref, out_i_ref, lg_ref):
    h = h_ref[...]
    w = w_ref[...]
    lg_ref[...] = jax.lax.dot_general(
        w, h, (((1,), (1,)), ((), ())), preferred_element_type=jnp.float32
    ) + b_ref[...]

    row = jax.lax.broadcasted_iota(jnp.int32, (NUM_EXPERTS, CHUNK), 0).astype(
        jnp.float32
    )
    krow = jax.lax.broadcasted_iota(jnp.int32, (TOP_K, CHUNK), 0)
    neg_inf = jnp.float32(-jnp.inf)
    big = jnp.float32(NUM_EXPERTS)

    def body(c, carry):
        sl = pl.ds(c * CHUNK, CHUNK)
        logits = lg_ref[:, sl]
        vals = jnp.zeros((TOP_K, CHUNK), dtype=jnp.float32)
        idxs = jnp.zeros((TOP_K, CHUNK), dtype=jnp.float32)
        for k in range(TOP_K):
            vmax = jnp.max(logits, axis=0, keepdims=True)
            imax = jnp.min(
                jnp.where(logits == vmax, row, big), axis=0, keepdims=True
            )
            vals = jnp.where(krow == k, vmax, vals)
            idxs = jnp.where(krow == k, imax, idxs)
            logits = jnp.where(row == imax, neg_inf, logits)
        e = jnp.exp(vals - vals[0:1, :])
        out_w_ref[:, sl] = e / jnp.sum(e, axis=0, keepdims=True)
        out_i_ref[:, sl] = idxs.astype(jnp.int32)
        return carry

    jax.lax.fori_loop(0, BLOCK_M // CHUNK, body, 0)


def kernel(hidden_states, weight, bias):
    n_tokens = hidden_states.shape[0]
    b = bias.reshape(NUM_EXPERTS, 1)
    grid = (n_tokens // BLOCK_M,)
    out_w, out_i = pl.pallas_call(
        _gate_kernel,
        grid=grid,
        in_specs=[
            pl.BlockSpec((BLOCK_M, D_MODEL), lambda i: (i, 0)),
            pl.BlockSpec((NUM_EXPERTS, D_MODEL), lambda i: (0, 0)),
            pl.BlockSpec((NUM_EXPERTS, 1), lambda i: (0, 0)),
        ],
        out_specs=[
            pl.BlockSpec((TOP_K, BLOCK_M), lambda i: (0, i)),
            pl.BlockSpec((TOP_K, BLOCK_M), lambda i: (0, i)),
        ],
        out_shape=[
            jax.ShapeDtypeStruct((TOP_K, n_tokens), jnp.float32),
            jax.ShapeDtypeStruct((TOP_K, n_tokens), jnp.int32),
        ],
        scratch_shapes=[pltpu.VMEM((NUM_EXPERTS, BLOCK_M), jnp.float32)],
    )(hidden_states, weight, b)
    return out_w.T, out_i.T


# speedup vs baseline: 1.1705x; 1.1705x over previous
import jax
import jax.numpy as jnp
from jax.experimental import pallas as pl
from jax.experimental.pallas import tpu as pltpu

TOP_K = 8
NUM_EXPERTS = 64
D_MODEL = 768
BLOCK_M = 4096
CHUNK = 512


def _gate_kernel(h_ref, w_ref, b_ref, out_w_ref, out_i_ref, lg_ref):
    h = h_ref[...]
    w = w_ref[...]
    lg_ref[...] = jax.lax.dot_general(
        w, h, (((1,), (1,)), ((), ())), preferred_element_type=jnp.float32
    ) + b_ref[...]

    row = jax.lax.broadcasted_iota(jnp.int32, (NUM_EXPERTS, CHUNK), 0).astype(
        jnp.float32
    )
    krow = jax.lax.broadcasted_iota(jnp.int32, (TOP_K, CHUNK), 0)
    neg_inf = jnp.float32(-jnp.inf)
    big = jnp.float32(NUM_EXPERTS)

    def body(c):
        sl = pl.ds(c * CHUNK, CHUNK)
        logits = lg_ref[:, sl]
        vals = jnp.zeros((TOP_K, CHUNK), dtype=jnp.float32)
        idxs = jnp.zeros((TOP_K, CHUNK), dtype=jnp.float32)
        for k in range(TOP_K):
            vmax = jnp.max(logits, axis=0, keepdims=True)
            imax = jnp.min(
                jnp.where(logits == vmax, row, big), axis=0, keepdims=True
            )
            vals = jnp.where(krow == k, vmax, vals)
            idxs = jnp.where(krow == k, imax, idxs)
            logits = jnp.where(row == imax, neg_inf, logits)
        e = jnp.exp(vals - vals[0:1, :])
        out_w_ref[:, sl] = e / jnp.sum(e, axis=0, keepdims=True)
        out_i_ref[:, sl] = idxs.astype(jnp.int32)

    for c in range(BLOCK_M // CHUNK):
        body(c)


def kernel(hidden_states, weight, bias):
    n_tokens = hidden_states.shape[0]
    b = bias.reshape(NUM_EXPERTS, 1)
    grid = (n_tokens // BLOCK_M,)
    out_w, out_i = pl.pallas_call(
        _gate_kernel,
        grid=grid,
        in_specs=[
            pl.BlockSpec((BLOCK_M, D_MODEL), lambda i: (i, 0)),
            pl.BlockSpec((NUM_EXPERTS, D_MODEL), lambda i: (0, 0)),
            pl.BlockSpec((NUM_EXPERTS, 1), lambda i: (0, 0)),
        ],
        out_specs=[
            pl.BlockSpec((TOP_K, BLOCK_M), lambda i: (0, i)),
            pl.BlockSpec((TOP_K, BLOCK_M), lambda i: (0, i)),
        ],
        out_shape=[
            jax.ShapeDtypeStruct((TOP_K, n_tokens), jnp.float32),
            jax.ShapeDtypeStruct((TOP_K, n_tokens), jnp.int32),
        ],
        scratch_shapes=[pltpu.VMEM((NUM_EXPERTS, BLOCK_M), jnp.float32)],
    )(hidden_states, weight, b)
    return out_w.T, out_i.T


# speedup vs baseline: 1.1725x; 1.0018x over previous
import jax
import jax.numpy as jnp
from jax.experimental import pallas as pl
from jax.experimental.pallas import tpu as pltpu

TOP_K = 8
NUM_EXPERTS = 64
D_MODEL = 768
BLOCK_M = 4096
CHUNK = 1024


def _gate_kernel(h_ref, w_ref, b_ref, out_w_ref, out_i_ref, lg_ref):
    h = h_ref[...]
    w = w_ref[...]
    lg_ref[...] = jax.lax.dot_general(
        w, h, (((1,), (1,)), ((), ())), preferred_element_type=jnp.float32
    ) + b_ref[...]

    row = jax.lax.broadcasted_iota(jnp.int32, (NUM_EXPERTS, CHUNK), 0).astype(
        jnp.float32
    )
    krow = jax.lax.broadcasted_iota(jnp.int32, (TOP_K, CHUNK), 0)
    neg_inf = jnp.float32(-jnp.inf)
    big = jnp.float32(NUM_EXPERTS)

    def body(c):
        sl = pl.ds(c * CHUNK, CHUNK)
        logits = lg_ref[:, sl]
        vals = jnp.zeros((TOP_K, CHUNK), dtype=jnp.float32)
        idxs = jnp.zeros((TOP_K, CHUNK), dtype=jnp.float32)
        for k in range(TOP_K):
            vmax = jnp.max(logits, axis=0, keepdims=True)
            imax = jnp.min(
                jnp.where(logits == vmax, row, big), axis=0, keepdims=True
            )
            vals = jnp.where(krow == k, vmax, vals)
            idxs = jnp.where(krow == k, imax, idxs)
            logits = jnp.where(row == imax, neg_inf, logits)
        e = jnp.exp(vals - vals[0:1, :])
        out_w_ref[:, sl] = e / jnp.sum(e, axis=0, keepdims=True)
        out_i_ref[:, sl] = idxs.astype(jnp.int32)

    for c in range(BLOCK_M // CHUNK):
        body(c)


def kernel(hidden_states, weight, bias):
    n_tokens = hidden_states.shape[0]
    b = bias.reshape(NUM_EXPERTS, 1)
    grid = (n_tokens // BLOCK_M,)
    out_w, out_i = pl.pallas_call(
        _gate_kernel,
        grid=grid,
        in_specs=[
            pl.BlockSpec((BLOCK_M, D_MODEL), lambda i: (i, 0)),
            pl.BlockSpec((NUM_EXPERTS, D_MODEL), lambda i: (0, 0)),
            pl.BlockSpec((NUM_EXPERTS, 1), lambda i: (0, 0)),
        ],
        out_specs=[
            pl.BlockSpec((TOP_K, BLOCK_M), lambda i: (0, i)),
            pl.BlockSpec((TOP_K, BLOCK_M), lambda i: (0, i)),
        ],
        out_shape=[
            jax.ShapeDtypeStruct((TOP_K, n_tokens), jnp.float32),
            jax.ShapeDtypeStruct((TOP_K, n_tokens), jnp.int32),
        ],
        scratch_shapes=[pltpu.VMEM((NUM_EXPERTS, BLOCK_M), jnp.float32)],
    )(hidden_states, weight, b)
    return out_w.T, out_i.T


# speedup vs baseline: 1.2353x; 1.0535x over previous
import jax
import jax.numpy as jnp
from jax.experimental import pallas as pl

TOP_K = 8
NUM_EXPERTS = 64
HALF = NUM_EXPERTS // 2
D_MODEL = 768
BLOCK_M = 4096


def _gate_kernel(h_ref, w_ref, b_ref, out_w_ref, out_i_ref):
    h = h_ref[...]
    w = w_ref[...]
    logits = jax.lax.dot_general(
        w, h, (((1,), (1,)), ((), ())), preferred_element_type=jnp.float32
    ) + b_ref[...]

    m = h.shape[0]
    row = jax.lax.broadcasted_iota(jnp.int32, (HALF, m), 0).astype(jnp.float32)
    krow = jax.lax.broadcasted_iota(jnp.int32, (TOP_K, m), 0)
    neg_inf = jnp.float32(-jnp.inf)
    big = jnp.float32(NUM_EXPERTS)

    top, bot = logits[:HALF, :], logits[HALF:, :]
    ge = top >= bot
    wv = jnp.where(ge, top, bot)
    wi = jnp.where(ge, row, row + HALF)
    lv = jnp.where(ge, bot, top)
    li = jnp.where(ge, row + HALF, row)

    vals = jnp.zeros((TOP_K, m), dtype=jnp.float32)
    idxs = jnp.zeros((TOP_K, m), dtype=jnp.float32)
    for k in range(TOP_K):
        vmax = jnp.max(wv, axis=0, keepdims=True)
        imax = jnp.min(
            jnp.where(wv == vmax, wi, big), axis=0, keepdims=True
        )
        vals = jnp.where(krow == k, vmax, vals)
        idxs = jnp.where(krow == k, imax, idxs)
        cond = wi == imax
        wv = jnp.where(cond, lv, wv)
        wi = jnp.where(cond, li, wi)
        lv = jnp.where(cond, neg_inf, lv)

    e = jnp.exp(vals - vals[0:1, :])
    out_w_ref[...] = e / jnp.sum(e, axis=0, keepdims=True)
    out_i_ref[...] = idxs.astype(jnp.int32)


def kernel(hidden_states, weight, bias):
    n_tokens = hidden_states.shape[0]
    b = bias.reshape(NUM_EXPERTS, 1)
    grid = (n_tokens // BLOCK_M,)
    out_w, out_i = pl.pallas_call(
        _gate_kernel,
        grid=grid,
        in_specs=[
            pl.BlockSpec((BLOCK_M, D_MODEL), lambda i: (i, 0)),
            pl.BlockSpec((NUM_EXPERTS, D_MODEL), lambda i: (0, 0)),
            pl.BlockSpec((NUM_EXPERTS, 1), lambda i: (0, 0)),
        ],
        out_specs=[
            pl.BlockSpec((TOP_K, BLOCK_M), lambda i: (0, i)),
            pl.BlockSpec((TOP_K, BLOCK_M), lambda i: (0, i)),
        ],
        out_shape=[
            jax.ShapeDtypeStruct((TOP_K, n_tokens), jnp.float32),
            jax.ShapeDtypeStruct((TOP_K, n_tokens), jnp.int32),
        ],
    )(hidden_states, weight, b)
    return out_w.T, out_i.T
